# baseline (speedup 1.0000x reference)
import functools

import jax
import jax.numpy as jnp
from jax import lax
from jax.experimental import pallas as pl
from jax.experimental.pallas import tpu as pltpu
from jax.experimental.pallas import tpu_sc as plsc

N_PAD = 5120
PER_W = N_PAD // 16
VREGS = PER_W // 16
ALL_VREGS = N_PAD // 16
MAX_OUT = 100


def _splat(x):
    return jnp.full((16,), x)


def _vperm(v, p):
    return lax.gather(
        v, p[:, None],
        dimension_numbers=lax.GatherDimensionNumbers(
            offset_dims=(), collapsed_slice_dims=(0,), start_index_map=(0,)),
        slice_sizes=(1,),
        mode=lax.GatherScatterMode.PROMISE_IN_BOUNDS)


def _amax_merge(m, mi, b, bi):
    take = jnp.logical_or(b > m, jnp.logical_and(b == m, bi < mi))
    return jnp.where(take, b, m), jnp.where(take, bi, mi)


def _nms_body(y1h, x1h, y2h, x2h, sh, outh,
              y1v, x1v, y2v, x2v, areav, sv,
              stage, table_sh, tablev, outv):
    cid = lax.axis_index("c")
    wid = lax.axis_index("s")
    base = wid * PER_W
    iota = lax.iota(jnp.int32, 16)

    pltpu.sync_copy(y1h, y1v)
    pltpu.sync_copy(x1h, x1v)
    pltpu.sync_copy(y2h, y2v)
    pltpu.sync_copy(x2h, x2v)
    pltpu.sync_copy(sh.at[pl.ds(base, PER_W)], sv)

    for j in range(ALL_VREGS):
        sl = pl.ds(j * 16, 16)
        areav[sl] = (y2v[sl] - y1v[sl]) * (x2v[sl] - x1v[sl])

    scores0 = [sv[pl.ds(j * 16, 16)] for j in range(VREGS)]

    perms = [jnp.bitwise_xor(iota, jnp.int32(p)) for p in (1, 2, 4, 8)]

    zero = jnp.zeros((16,), jnp.float32)

    def body(t, carry):
        py1, px1, py2, px2, pa = carry[:5]
        scores = carry[5:]

        best = jnp.full((16,), -2.0)
        bidx = jnp.zeros((16,), jnp.int32)
        idxv = base + iota
        new_scores = []
        for j in range(VREGS):
            sl = pl.ds(base + j * 16, 16)
            iy1 = jnp.maximum(py1, y1v[sl])
            ix1 = jnp.maximum(px1, x1v[sl])
            iy2 = jnp.minimum(py2, y2v[sl])
            ix2 = jnp.minimum(px2, x2v[sl])
            inter = jnp.maximum(iy2 - iy1, 0.0) * jnp.maximum(ix2 - ix1, 0.0)
            union = pa + areav[sl] - inter
            s = jnp.where(inter + inter > union, -1.0, scores[j])
            new_scores.append(s)
            gt = s > best
            best = jnp.where(gt, s, best)
            bidx = jnp.where(gt, idxv, bidx)
            idxv = idxv + 16

        stage[pl.ds(0, 16)] = best
        stage[pl.ds(16, 16)] = plsc.bitcast(bidx, jnp.float32)
        off = (t & 1) * (16 * 32)
        pltpu.sync_copy(stage, table_sh.at[pl.ds(off + wid * 32, 32)])
        plsc.subcore_barrier()
        pltpu.sync_copy(table_sh.at[pl.ds(off, 16 * 32)], tablev)

        ms = [tablev[pl.ds(w * 32, 16)] for w in range(16)]
        mis = [plsc.bitcast(tablev[pl.ds(w * 32 + 16, 16)], jnp.int32)
               for w in range(16)]
        width = 16
        while width > 1:
            width //= 2
            for w in range(width):
                ms[w], mis[w] = _amax_merge(ms[w], mis[w],
                                            ms[w + width], mis[w + width])
        m, mi = ms[0], mis[0]
        for p in perms:
            m2 = _vperm(m, p)
            mi2 = _vperm(mi, p)
            m, mi = _amax_merge(m, mi, m2, mi2)

        npy1 = plsc.load_gather(y1v, [mi])
        npx1 = plsc.load_gather(x1v, [mi])
        npy2 = plsc.load_gather(y2v, [mi])
        npx2 = plsc.load_gather(x2v, [mi])
        npa = plsc.load_gather(areav, [mi])

        hasf = (m >= 0.0).astype(jnp.float32)

        @pl.when(jnp.logical_and(cid == 0, wid == 0))
        def _():
            v = jnp.where(iota == 0, npy1,
                jnp.where(iota == 1, npx1,
                jnp.where(iota == 2, npy2, npx2))) * hasf
            plsc.store_scatter(outv, [t * 4 + iota], v, mask=iota < 4)

        return (npy1, npx1, npy2, npx2, npa, *new_scores)

    lax.fori_loop(0, MAX_OUT, body, (zero, zero, zero, zero, zero, *scores0),
                  unroll=False)

    @pl.when(jnp.logical_and(cid == 0, wid == 0))
    def _():
        pltpu.sync_copy(outv.at[pl.ds(0, MAX_OUT * 4)], outh)


@jax.jit
def _nms(y1, x1, y2, x2, s):
    mesh = plsc.VectorSubcoreMesh(core_axis_name="c", subcore_axis_name="s")
    f = functools.partial(
        pl.kernel,
        mesh=mesh,
        compiler_params=pltpu.CompilerParams(needs_layout_passes=False),
        out_type=jax.ShapeDtypeStruct((MAX_OUT * 4,), jnp.float32),
        scratch_types=[
            pltpu.VMEM((N_PAD,), jnp.float32),
            pltpu.VMEM((N_PAD,), jnp.float32),
            pltpu.VMEM((N_PAD,), jnp.float32),
            pltpu.VMEM((N_PAD,), jnp.float32),
            pltpu.VMEM((N_PAD,), jnp.float32),
            pltpu.VMEM((PER_W,), jnp.float32),
            pltpu.VMEM((32,), jnp.float32),
            pltpu.VMEM_SHARED((2 * 16 * 32,), jnp.float32),
            pltpu.VMEM((16 * 32,), jnp.float32),
            pltpu.VMEM((MAX_OUT * 4 + 16,), jnp.float32),
        ],
    )(_nms_body)
    return f(y1, x1, y2, x2, s)


def kernel(boxes, scores, max_output_size):
    n = boxes.shape[0]
    pad = N_PAD - n
    y1 = jnp.pad(boxes[:, 0], (0, pad))
    x1 = jnp.pad(boxes[:, 1], (0, pad))
    y2 = jnp.pad(boxes[:, 2], (0, pad))
    x2 = jnp.pad(boxes[:, 3], (0, pad))
    s = jnp.pad(scores, (0, pad), constant_values=-1.0)
    out = _nms(y1, x1, y2, x2, s).reshape(MAX_OUT, 4)
    keep = (lax.iota(jnp.int32, MAX_OUT) < max_output_size)[:, None]
    return jnp.where(keep, out, 0.0)
